# baseline (speedup 1.0000x reference)
import functools

import jax
import jax.numpy as jnp
from jax.experimental import pallas as pl
from jax.experimental.pallas import tpu as pltpu
from jax.experimental.pallas import tpu_sc as plsc

_R = 4


def _sc_gather(table, idx):
    b, s = idx.shape
    d = table.shape[1]
    mesh = plsc.VectorSubcoreMesh(core_axis_name="c", subcore_axis_name="s")

    @functools.partial(
        pl.kernel,
        out_type=jax.ShapeDtypeStruct((b, s, d), table.dtype),
        mesh=mesh,
        compiler_params=pltpu.CompilerParams(use_tc_tiling_on_sc=False),
    )
    def gather_kernel(table_hbm, idx_hbm, out_hbm):
        def body(idx_vmem, out_vmem):
            for r in range(_R):
                pltpu.sync_copy(table_hbm.at[idx_vmem.at[r]], out_vmem.at[r])

        pltpu.emit_pipeline(
            body,
            grid=(b // _R,),
            in_specs=[pl.BlockSpec((_R, s), index_map=lambda i: (i, 0))],
            out_specs=[pl.BlockSpec((_R, s, d), index_map=lambda i: (i, 0, 0))],
            core_axis_name=("c", "s"),
            dimension_semantics=(pltpu.PARALLEL,),
        )(idx_hbm, out_hbm)

    return gather_kernel(table, idx)


def kernel(x, embed_weight):
    return _sc_gather(embed_weight, x.astype(jnp.int32))


# speedup vs baseline: 1.0446x; 1.0446x over previous
import functools

import jax
import jax.numpy as jnp
from jax.experimental import pallas as pl
from jax.experimental.pallas import tpu as pltpu
from jax.experimental.pallas import tpu_sc as plsc

_C = 8


def _sc_gather(table, idx_t):
    s, b = idx_t.shape
    d = table.shape[1]
    w = b // _C
    mesh = plsc.VectorSubcoreMesh(core_axis_name="c", subcore_axis_name="s")

    @functools.partial(
        pl.kernel,
        out_type=jax.ShapeDtypeStruct((s, b, d), table.dtype),
        mesh=mesh,
        compiler_params=pltpu.CompilerParams(use_tc_tiling_on_sc=False),
    )
    def gather_kernel(table_hbm, idx_hbm, out_hbm):
        def body(idx_vmem, out_vmem):
            pltpu.sync_copy(table_hbm.at[idx_vmem.at[0]], out_vmem.at[0])

        pltpu.emit_pipeline(
            body,
            grid=(s, _C),
            in_specs=[pl.BlockSpec((1, w), index_map=lambda i, j: (i, j))],
            out_specs=[pl.BlockSpec((1, w, d), index_map=lambda i, j: (i, j, 0))],
            core_axis_name=("c", "s"),
            dimension_semantics=(pltpu.PARALLEL, pltpu.PARALLEL),
        )(idx_hbm, out_hbm)

    return gather_kernel(table, idx_t)


def kernel(x, embed_weight):
    out = _sc_gather(embed_weight, x.T.astype(jnp.int32))
    return out.transpose(1, 0, 2)


# speedup vs baseline: 1.3080x; 1.2522x over previous
import functools

import jax
import jax.numpy as jnp
from jax import lax
from jax.experimental import pallas as pl
from jax.experimental.pallas import tpu as pltpu
from jax.experimental.pallas import tpu_sc as plsc

_NW = 32


def _sc_gather_native(tt, xt):
    d_model, v = tt.shape
    s_len, b = xt.shape
    max_rows = (s_len + _NW - 1) // _NW
    full_upto = s_len - (max_rows - 1) * _NW
    mesh = plsc.VectorSubcoreMesh(core_axis_name="c", subcore_axis_name="s")

    @functools.partial(
        pl.kernel,
        out_type=jax.ShapeDtypeStruct((s_len, d_model, b), tt.dtype),
        mesh=mesh,
        scratch_types=[
            [pltpu.VMEM((b,), jnp.int32) for _ in range(max_rows)],
            [pltpu.VMEM((b,), tt.dtype) for _ in range(max_rows)],
            pltpu.VMEM_SHARED((v,), tt.dtype),
            pltpu.SemaphoreType.DMA,
            pltpu.SemaphoreType.DMA,
        ],
    )
    def k(tt_hbm, xt_hbm, out_hbm, idx_bufs, val_bufs, row0, sem_a, sem_o):
        cid = lax.axis_index("c")
        sid = lax.axis_index("s")
        wid = cid * 16 + sid
        n_rows = jnp.where(wid < full_upto, max_rows, max_rows - 1)

        for j in range(max_rows):
            @pl.when(j < n_rows)
            def _():
                pltpu.sync_copy(xt_hbm.at[wid + j * _NW], idx_bufs[j])

        @pl.when(sid == 0)
        def _():
            pltpu.async_copy(tt_hbm.at[0], row0, sem_a)

        def do_dim(d, row, sem):
            @pl.when(sid == 0)
            def _():
                pltpu.make_async_copy(tt_hbm.at[d], row, sem).wait()

            plsc.subcore_barrier()

            for j in range(max_rows):
                @pl.when(j < n_rows)
                def _():
                    pltpu.sync_copy(row.at[idx_bufs[j]], val_bufs[j])
                    pltpu.async_copy(val_bufs[j], out_hbm.at[wid + j * _NW, d], sem_o)

            for j in range(max_rows):
                @pl.when(j < n_rows)
                def _():
                    pltpu.make_async_copy(val_bufs[j], out_hbm.at[wid + j * _NW, d], sem_o).wait()

            plsc.subcore_barrier()

            @pl.when((sid == 0) & (d + 1 < d_model))
            def _():
                pltpu.async_copy(tt_hbm.at[d + 1], row, sem)

        @pl.loop(0, d_model)
        def _(d):
            do_dim(d, row0, sem_a)

    return k(tt, xt)


def kernel(x, embed_weight):
    out = _sc_gather_native(embed_weight.T, x.T.astype(jnp.int32))
    return out.transpose(2, 0, 1)


# speedup vs baseline: 1.3844x; 1.0584x over previous
import functools

import jax
import jax.numpy as jnp
from jax import lax
from jax.experimental import pallas as pl
from jax.experimental.pallas import tpu as pltpu
from jax.experimental.pallas import tpu_sc as plsc

_NW = 32


def _sc_gather_native(tt, xt):
    d_model, v = tt.shape
    s_len, b = xt.shape
    max_rows = (s_len + _NW - 1) // _NW
    full_upto = s_len - (max_rows - 1) * _NW
    mesh = plsc.VectorSubcoreMesh(core_axis_name="c", subcore_axis_name="s")

    @functools.partial(
        pl.kernel,
        out_type=jax.ShapeDtypeStruct((s_len, d_model, b), tt.dtype),
        mesh=mesh,
        scratch_types=[
            pltpu.VMEM((max_rows * b,), jnp.int32),
            pltpu.VMEM((max_rows * b,), tt.dtype),
            pltpu.VMEM_SHARED((v,), tt.dtype),
            pltpu.SemaphoreType.DMA,
            pltpu.SemaphoreType.DMA,
        ],
    )
    def k(tt_hbm, xt_hbm, out_hbm, idx_v, val0, row, sem_a, sem_o):
        cid = lax.axis_index("c")
        sid = lax.axis_index("s")
        wid = cid * 16 + sid
        n_rows = jnp.where(wid < full_upto, max_rows, max_rows - 1)

        for j in range(max_rows):
            src_row = jnp.where(j < n_rows, wid + j * _NW, 0)
            pltpu.sync_copy(xt_hbm.at[src_row], idx_v.at[pl.ds(j * b, b)])

        @pl.when(sid == 0)
        def _():
            pltpu.async_copy(tt_hbm.at[0], row, sem_a)

        def fire_writes(d, val):
            for j in range(max_rows):
                @pl.when(j < n_rows)
                def _():
                    pltpu.async_copy(
                        val.at[pl.ds(j * b, b)], out_hbm.at[wid + j * _NW, d], sem_o
                    )

        def drain_writes(d, val):
            for j in range(max_rows):
                @pl.when(j < n_rows)
                def _():
                    pltpu.make_async_copy(
                        val.at[pl.ds(j * b, b)], out_hbm.at[wid + j * _NW, d], sem_o
                    ).wait()

        def do_dim(d, val, first):
            @pl.when(sid == 0)
            def _():
                pltpu.make_async_copy(tt_hbm.at[d], row, sem_a).wait()

            plsc.subcore_barrier()

            if not first:
                drain_writes(d - 1, val)

            pltpu.sync_copy(row.at[idx_v], val)

            plsc.subcore_barrier()

            @pl.when((sid == 0) & (d + 1 < d_model))
            def _():
                pltpu.async_copy(tt_hbm.at[d + 1], row, sem_a)

            fire_writes(d, val)

        do_dim(0, val0, True)

        @pl.loop(1, d_model)
        def _(d):
            do_dim(d, val0, False)

        drain_writes(d_model - 1, val0)

    return k(tt, xt)


def kernel(x, embed_weight):
    out = _sc_gather_native(embed_weight.T, x.T.astype(jnp.int32))
    return out.transpose(2, 0, 1)


# speedup vs baseline: 1.3859x; 1.0011x over previous
import functools

import jax
import jax.numpy as jnp
from jax import lax
from jax.experimental import pallas as pl
from jax.experimental.pallas import tpu as pltpu
from jax.experimental.pallas import tpu_sc as plsc

_NW = 32


def _sc_gather_native(tt, xt):
    d_model, v = tt.shape
    s_len, b = xt.shape
    max_rows = (s_len + _NW - 1) // _NW
    full_upto = s_len - (max_rows - 1) * _NW
    rows_a = (max_rows + 1) // 2
    rows_b = max_rows - rows_a
    mesh = plsc.VectorSubcoreMesh(core_axis_name="c", subcore_axis_name="s")

    @functools.partial(
        pl.kernel,
        out_type=jax.ShapeDtypeStruct((s_len, d_model, b), tt.dtype),
        mesh=mesh,
        scratch_types=[
            pltpu.VMEM((rows_a * b,), jnp.int32),
            pltpu.VMEM((rows_b * b,), jnp.int32),
            pltpu.VMEM((rows_a * b,), tt.dtype),
            pltpu.VMEM((rows_b * b,), tt.dtype),
            pltpu.VMEM_SHARED((v,), tt.dtype),
            pltpu.SemaphoreType.DMA,
            pltpu.SemaphoreType.DMA,
            pltpu.SemaphoreType.DMA,
        ],
    )
    def k(tt_hbm, xt_hbm, out_hbm, idx_a, idx_b, val_a, val_b, row, sem_a, sem_g, sem_o):
        cid = lax.axis_index("c")
        sid = lax.axis_index("s")
        wid = cid * 16 + sid
        n_rows = jnp.where(wid < full_upto, max_rows, max_rows - 1)

        for j in range(max_rows):
            src_row = jnp.where(j < n_rows, wid + j * _NW, 0)
            if j < rows_a:
                dst = idx_a.at[pl.ds(j * b, b)]
            else:
                dst = idx_b.at[pl.ds((j - rows_a) * b, b)]
            pltpu.sync_copy(xt_hbm.at[src_row], dst)

        @pl.when(sid == 0)
        def _():
            pltpu.async_copy(tt_hbm.at[0], row, sem_a)

        def val_slice(j):
            if j < rows_a:
                return val_a.at[pl.ds(j * b, b)]
            return val_b.at[pl.ds((j - rows_a) * b, b)]

        def fire_writes(d):
            for j in range(max_rows):
                @pl.when(j < n_rows)
                def _():
                    pltpu.async_copy(val_slice(j), out_hbm.at[wid + j * _NW, d], sem_o)

        def drain_writes(d):
            for j in range(max_rows):
                @pl.when(j < n_rows)
                def _():
                    pltpu.make_async_copy(
                        val_slice(j), out_hbm.at[wid + j * _NW, d], sem_o
                    ).wait()

        def do_dim(d, first):
            @pl.when(sid == 0)
            def _():
                pltpu.make_async_copy(tt_hbm.at[d], row, sem_a).wait()

            plsc.subcore_barrier()

            if not first:
                drain_writes(d - 1)

            pltpu.async_copy(row.at[idx_a], val_a, sem_g)
            pltpu.async_copy(row.at[idx_b], val_b, sem_g)
            pltpu.make_async_copy(row.at[idx_a], val_a, sem_g).wait()
            pltpu.make_async_copy(row.at[idx_b], val_b, sem_g).wait()

            plsc.subcore_barrier()

            @pl.when((sid == 0) & (d + 1 < d_model))
            def _():
                pltpu.async_copy(tt_hbm.at[d + 1], row, sem_a)

            fire_writes(d)

        do_dim(0, True)

        @pl.loop(1, d_model)
        def _(d):
            do_dim(d, False)

        drain_writes(d_model - 1)

    return k(tt, xt)


def kernel(x, embed_weight):
    out = _sc_gather_native(embed_weight.T, x.T.astype(jnp.int32))
    return out.transpose(2, 0, 1)


# speedup vs baseline: 1.4850x; 1.0715x over previous
import functools

import jax
import jax.numpy as jnp
from jax import lax
from jax.experimental import pallas as pl
from jax.experimental.pallas import tpu as pltpu
from jax.experimental.pallas import tpu_sc as plsc

_NW = 32


def _sc_gather_native(tt, xt):
    d_model, v = tt.shape
    s_len, b = xt.shape
    n_full = s_len // _NW
    tail_s0 = n_full * _NW
    n_tail = s_len - tail_s0
    tw = n_tail * b // _NW
    per_tail = b // tw
    total = n_full * b + tw
    mesh = plsc.VectorSubcoreMesh(core_axis_name="c", subcore_axis_name="s")

    @functools.partial(
        pl.kernel,
        out_type=jax.ShapeDtypeStruct((s_len, d_model, b), tt.dtype),
        mesh=mesh,
        scratch_types=[
            pltpu.VMEM((total,), jnp.int32),
            pltpu.VMEM((total,), tt.dtype),
            pltpu.VMEM_SHARED((v,), tt.dtype),
            pltpu.SemaphoreType.DMA,
            pltpu.SemaphoreType.DMA,
        ],
    )
    def k(tt_hbm, xt_hbm, out_hbm, idx_v, val, row, sem_a, sem_o):
        cid = lax.axis_index("c")
        sid = lax.axis_index("s")
        wid = cid * 16 + sid
        tail_s = tail_s0 + wid // per_tail
        tail_q = (wid % per_tail) * tw

        for j in range(n_full):
            pltpu.sync_copy(xt_hbm.at[wid + j * _NW], idx_v.at[pl.ds(j * b, b)])
        pltpu.sync_copy(
            xt_hbm.at[tail_s, pl.ds(tail_q, tw)],
            idx_v.at[pl.ds(n_full * b, tw)],
        )

        @pl.when(sid == 0)
        def _():
            pltpu.async_copy(tt_hbm.at[0], row, sem_a)

        def writes(d, fire):
            for j in range(n_full):
                args = (val.at[pl.ds(j * b, b)], out_hbm.at[wid + j * _NW, d], sem_o)
                if fire:
                    pltpu.async_copy(*args)
                else:
                    pltpu.make_async_copy(*args).wait()
            args = (
                val.at[pl.ds(n_full * b, tw)],
                out_hbm.at[tail_s, d, pl.ds(tail_q, tw)],
                sem_o,
            )
            if fire:
                pltpu.async_copy(*args)
            else:
                pltpu.make_async_copy(*args).wait()

        def do_dim(d, first):
            @pl.when(sid == 0)
            def _():
                pltpu.make_async_copy(tt_hbm.at[d], row, sem_a).wait()

            plsc.subcore_barrier()

            if not first:
                writes(d - 1, fire=False)

            pltpu.sync_copy(row.at[idx_v], val)

            plsc.subcore_barrier()

            @pl.when((sid == 0) & (d + 1 < d_model))
            def _():
                pltpu.async_copy(tt_hbm.at[d + 1], row, sem_a)

            writes(d, fire=True)

        do_dim(0, True)

        @pl.loop(1, d_model)
        def _(d):
            do_dim(d, False)

        writes(d_model - 1, fire=False)

    return k(tt, xt)


def kernel(x, embed_weight):
    out = _sc_gather_native(embed_weight.T, x.T.astype(jnp.int32))
    return out.transpose(2, 0, 1)
